# Initial kernel scaffold; baseline (speedup 1.0000x reference)
import jax
import jax.numpy as jnp
from jax.experimental import pallas as pl


def kernel(tokens, W):
    raise NotImplementedError("write your pallas kernel here")


# speedup vs baseline: 1.8819x; 1.8819x over previous
import functools

import jax
import jax.numpy as jnp
from jax.experimental import pallas as pl
from jax.experimental.pallas import tpu as pltpu

N_GROUP = 8
TOPK_GROUP = 4
TOP_K = 8


def _router_body(grid_n, tok_ref, wt_ref, idx_ref, w_ref, aux_ref, acc_ref):
    i = pl.program_id(0)
    B = tok_ref.shape[0]
    E = wt_ref.shape[1]
    eg = E // N_GROUP

    logits = jnp.dot(tok_ref[...], wt_ref[...], preferred_element_type=jnp.float32)
    scores = jax.nn.sigmoid(logits)

    col = jax.lax.broadcasted_iota(jnp.int32, (B, E), 1)
    gid = col // eg

    NEG = jnp.float32(-1.0)
    gfull = jnp.zeros_like(scores)
    for g in range(N_GROUP):
        sub = jnp.where(gid == g, scores, NEG)
        m = jnp.max(sub, axis=-1, keepdims=True)
        gfull = jnp.where(gid == g, m, gfull)

    gwork = gfull
    group_sel = jnp.zeros((B, E), jnp.bool_)
    for _ in range(TOPK_GROUP):
        m = jnp.max(gwork, axis=-1, keepdims=True)
        sel_g = jnp.min(jnp.where(gwork == m, gid, N_GROUP), axis=-1, keepdims=True)
        hit = gid == sel_g
        group_sel = group_sel | hit
        gwork = jnp.where(hit, NEG, gwork)

    routed = jnp.where(group_sel, scores, jnp.float32(0.0))

    sel = jnp.zeros((B, E), jnp.bool_)
    work = routed
    idxs = []
    ws = []
    for _ in range(TOP_K):
        m = jnp.max(work, axis=-1, keepdims=True)
        sidx = jnp.min(jnp.where(work == m, col, E), axis=-1, keepdims=True)
        hit = col == sidx
        sel = sel | hit
        work = jnp.where(hit, NEG, work)
        idxs.append(sidx)
        ws.append(m)
    topi = jnp.concatenate(idxs, axis=1)
    topw = jnp.concatenate(ws, axis=1)
    denom = jnp.maximum(jnp.sum(topw, axis=1, keepdims=True), jnp.float32(1e-9))
    idx_ref[...] = topi
    w_ref[...] = topw / denom

    ssum = jnp.maximum(jnp.sum(scores, axis=-1, keepdims=True), jnp.float32(1e-9))
    p_norm = jnp.sum(scores / ssum, axis=0, keepdims=True)
    p_cnt = jnp.sum(jnp.where(sel, jnp.float32(1.0), jnp.float32(0.0)),
                    axis=0, keepdims=True)
    part = jnp.concatenate([p_norm, p_cnt], axis=0)

    @pl.when(i == 0)
    def _init():
        acc_ref[...] = jnp.zeros_like(acc_ref)

    acc_ref[...] += part

    @pl.when(i == grid_n - 1)
    def _fin():
        T = grid_n * B
        a = acc_ref[...]
        scale = jnp.float32(E) / (jnp.float32(T) * jnp.float32(T) * jnp.float32(TOP_K))
        aux = jnp.sum(a[0:1, :] * a[1:2, :]) * scale
        aux_ref[...] = jnp.full((1, 1), aux, jnp.float32)


def kernel(tokens, W):
    T, H = tokens.shape
    E = W.shape[0]
    B = 256
    grid_n = T // B
    Wt = W.T

    idx, w, aux = pl.pallas_call(
        functools.partial(_router_body, grid_n),
        grid=(grid_n,),
        in_specs=[
            pl.BlockSpec((B, H), lambda i: (i, 0)),
            pl.BlockSpec((H, E), lambda i: (0, 0)),
        ],
        out_specs=[
            pl.BlockSpec((B, TOP_K), lambda i: (i, 0)),
            pl.BlockSpec((B, TOP_K), lambda i: (i, 0)),
            pl.BlockSpec((1, 1), lambda i: (0, 0)),
        ],
        out_shape=[
            jax.ShapeDtypeStruct((T, TOP_K), jnp.int32),
            jax.ShapeDtypeStruct((T, TOP_K), jnp.float32),
            jax.ShapeDtypeStruct((1, 1), jnp.float32),
        ],
        scratch_shapes=[pltpu.VMEM((2, E), jnp.float32)],
        compiler_params=pltpu.CompilerParams(
            dimension_semantics=("arbitrary",),
        ),
    )(tokens, Wt)
    return (idx, w, aux[0, 0])


# speedup vs baseline: 4.2288x; 2.2470x over previous
import functools

import jax
import jax.numpy as jnp
from jax.experimental import pallas as pl
from jax.experimental.pallas import tpu as pltpu

N_GROUP = 8
TOPK_GROUP = 4
TOP_K = 8


def _router_body(grid_n, tok_ref, wt_ref, idx_ref, w_ref, aux_ref,
                 accn_ref, accc_ref):
    i = pl.program_id(0)
    B = tok_ref.shape[0]
    E = wt_ref.shape[1]
    eg = E // N_GROUP
    NEG = jnp.float32(-1.0)

    logits = jnp.dot(tok_ref[...], wt_ref[...], preferred_element_type=jnp.float32)
    scores = jax.nn.sigmoid(logits.T)

    gm = jnp.concatenate(
        [jnp.max(scores[g * eg:(g + 1) * eg], axis=0, keepdims=True)
         for g in range(N_GROUP)], axis=0)

    grow = jax.lax.broadcasted_iota(jnp.int32, (N_GROUP, B), 0)
    gwork = gm
    gsel = jnp.zeros((N_GROUP, B), jnp.bool_)
    for _ in range(TOPK_GROUP):
        m = jnp.max(gwork, axis=0, keepdims=True)
        selg = jnp.min(jnp.where(gwork == m, grow, N_GROUP), axis=0, keepdims=True)
        hit = grow == selg
        gsel = gsel | hit
        gwork = jnp.where(hit, NEG, gwork)

    routed = jnp.concatenate(
        [jnp.where(gsel[g:g + 1], scores[g * eg:(g + 1) * eg], jnp.float32(0.0))
         for g in range(N_GROUP)], axis=0)

    erow = jax.lax.broadcasted_iota(jnp.int32, (E, B), 0)
    work = routed
    sel = jnp.zeros((E, B), jnp.bool_)
    idxs = []
    ws = []
    for _ in range(TOP_K):
        m = jnp.max(work, axis=0, keepdims=True)
        sidx = jnp.min(jnp.where(work == m, erow, E), axis=0, keepdims=True)
        hit = erow == sidx
        sel = sel | hit
        work = jnp.where(hit, NEG, work)
        idxs.append(sidx)
        ws.append(m)
    topi = jnp.concatenate(idxs, axis=0)
    topw = jnp.concatenate(ws, axis=0)
    denom = jnp.maximum(jnp.sum(topw, axis=0, keepdims=True), jnp.float32(1e-9))
    idx_ref[...] = topi
    w_ref[...] = topw / denom

    ssum = jnp.maximum(jnp.sum(scores, axis=0, keepdims=True), jnp.float32(1e-9))
    normed = scores / ssum

    @pl.when(i == 0)
    def _init():
        accn_ref[...] = jnp.zeros_like(accn_ref)
        accc_ref[...] = jnp.zeros_like(accc_ref)

    accn_ref[...] += normed
    accc_ref[...] += jnp.where(sel, jnp.float32(1.0), jnp.float32(0.0))

    @pl.when(i == grid_n - 1)
    def _fin():
        T = grid_n * B
        an = jnp.sum(accn_ref[...], axis=1, keepdims=True)
        ac = jnp.sum(accc_ref[...], axis=1, keepdims=True)
        scale = jnp.float32(E) / (jnp.float32(T) * jnp.float32(T) * jnp.float32(TOP_K))
        aux_ref[...] = jnp.full((1, 1), jnp.sum(an * ac) * scale, jnp.float32)


def kernel(tokens, W):
    T, H = tokens.shape
    E = W.shape[0]
    B = 256
    grid_n = T // B
    Wt = W.T

    idx, w, aux = pl.pallas_call(
        functools.partial(_router_body, grid_n),
        grid=(grid_n,),
        in_specs=[
            pl.BlockSpec((B, H), lambda i: (i, 0)),
            pl.BlockSpec((H, E), lambda i: (0, 0)),
        ],
        out_specs=[
            pl.BlockSpec((TOP_K, B), lambda i: (0, i)),
            pl.BlockSpec((TOP_K, B), lambda i: (0, i)),
            pl.BlockSpec((1, 1), lambda i: (0, 0)),
        ],
        out_shape=[
            jax.ShapeDtypeStruct((TOP_K, T), jnp.int32),
            jax.ShapeDtypeStruct((TOP_K, T), jnp.float32),
            jax.ShapeDtypeStruct((1, 1), jnp.float32),
        ],
        scratch_shapes=[
            pltpu.VMEM((E, B), jnp.float32),
            pltpu.VMEM((E, B), jnp.float32),
        ],
        compiler_params=pltpu.CompilerParams(
            dimension_semantics=("arbitrary",),
        ),
    )(tokens, Wt)
    return (idx.T, w.T, aux[0, 0])
